# baseline (speedup 1.0000x reference)
import functools

import jax
import jax.numpy as jnp
from jax import lax
from jax.experimental import pallas as pl
from jax.experimental.pallas import tpu as pltpu
from jax.experimental.pallas import tpu_sc as plsc

_NUM_WORKERS = 32
_CHUNK = 1024
_NBUF = 3


def _gather_kernel(n_rows, emb):
  per_w = n_rows // _NUM_WORKERS
  n_chunks = per_w // _CHUNK
  mesh = plsc.VectorSubcoreMesh(core_axis_name="c", subcore_axis_name="s")

  @functools.partial(
      pl.kernel,
      mesh=mesh,
      out_type=jax.ShapeDtypeStruct((n_rows, emb), jnp.float32),
      scratch_types=[
          pltpu.VMEM((_NBUF, _CHUNK), jnp.int32),
          pltpu.VMEM((_NBUF, _CHUNK, emb), jnp.float32),
          [pltpu.SemaphoreType.DMA] * _NBUF,
          [pltpu.SemaphoreType.DMA] * _NBUF,
          [pltpu.SemaphoreType.DMA] * _NBUF,
      ],
      compiler_params=pltpu.CompilerParams(use_tc_tiling_on_sc=False),
  )
  def k(idx_hbm, table_hbm, out_hbm, idx_v, rows_v, si, sg, sw):
    wid = lax.axis_index("s") * 2 + lax.axis_index("c")
    base = wid * per_w

    def fire_idx(i):
      b = i % _NBUF
      return pltpu.async_copy(idx_hbm.at[pl.ds(base + i * _CHUNK, _CHUNK)],
                              idx_v.at[b], si[b])

    def fire_gather(b):
      return pltpu.async_copy(table_hbm.at[idx_v.at[b]], rows_v.at[b], sg[b])

    idx_h = [None] * _NBUF
    g = [None] * _NBUF
    w = [None] * _NBUF

    idx_h[0] = fire_idx(0)
    if n_chunks > 1:
      idx_h[1] = fire_idx(1)
    idx_h[0].wait()
    g[0] = fire_gather(0)

    for i in range(n_chunks):
      cur = i % _NBUF
      nxt = (i + 1) % _NBUF
      if i + 2 < n_chunks:
        idx_h[(i + 2) % _NBUF] = fire_idx(i + 2)
      if i + 1 < n_chunks:
        idx_h[nxt].wait()
        if w[nxt] is not None:
          w[nxt].wait()
          w[nxt] = None
        g[nxt] = fire_gather(nxt)
      g[cur].wait()
      w[cur] = pltpu.async_copy(
          rows_v.at[cur], out_hbm.at[pl.ds(base + i * _CHUNK, _CHUNK)],
          sw[cur])

    for b in range(_NBUF):
      if w[b] is not None:
        w[b].wait()

  return k


def kernel(tokens, table):
  batch, seq = tokens.shape
  vocab, emb = table.shape
  n_rows = batch * seq
  flat = tokens.reshape(n_rows).astype(jnp.int32)
  out = _gather_kernel(n_rows, emb)(flat, table)
  return out.reshape(batch, seq, emb)
